# Initial kernel scaffold; baseline (speedup 1.0000x reference)
import jax
import jax.numpy as jnp
from jax.experimental import pallas as pl


def kernel(father, mask):
    raise NotImplementedError("write your pallas kernel here")


# speedup vs baseline: 10.4873x; 10.4873x over previous
import functools

import jax
import jax.numpy as jnp
from jax import lax
from jax.experimental import pallas as pl
from jax.experimental.pallas import tpu as pltpu
from jax.experimental.pallas import tpu_sc as plsc

N = 16777216
B = 1048576

NC = 2
NS = 16
NW = NC * NS
BPW = B // NW
CHUNK = 128
NCHUNK = BPW // CHUNK
L = 16


def _dsu_body(father_hbm, mask_hbm, out_hbm, r_ref, g_ref, sem):
    wid = lax.axis_index("s") * NC + lax.axis_index("c")
    pltpu.sync_copy(mask_hbm.at[wid], r_ref)

    def fire(j, carry):
        pltpu.make_async_copy(
            father_hbm.at[r_ref.at[pl.ds(j * CHUNK, CHUNK)]],
            g_ref.at[pl.ds(j * CHUNK, CHUNK)],
            sem,
        ).start()
        return carry

    def cmp_update(i, acc):
        g16 = g_ref[pl.ds(i * L, L)]
        r16 = r_ref[pl.ds(i * L, L)]
        r_ref[pl.ds(i * L, L)] = g16
        return acc | (g16 ^ r16)

    def body(_):
        lax.fori_loop(0, NCHUNK, fire, 0)
        pltpu.make_async_copy(father_hbm.at[pl.ds(0, BPW)], g_ref, sem).wait()
        acc = lax.fori_loop(0, BPW // L, cmp_update, jnp.zeros((L,), jnp.int32))
        return jnp.max(acc)

    lax.while_loop(lambda ch: ch != 0, body, jnp.int32(1))
    pltpu.sync_copy(r_ref, out_hbm.at[wid])


_call = functools.partial(
    pl.kernel,
    out_type=jax.ShapeDtypeStruct((NW, BPW), jnp.int32),
    mesh=plsc.VectorSubcoreMesh(core_axis_name="c", subcore_axis_name="s"),
    scratch_types=[
        pltpu.VMEM((BPW,), jnp.int32),
        pltpu.VMEM((BPW,), jnp.int32),
        pltpu.SemaphoreType.DMA,
    ],
    compiler_params=pltpu.CompilerParams(needs_layout_passes=False),
)(_dsu_body)


def kernel(father, mask):
    out = _call(father, mask.reshape(NW, BPW))
    return out.reshape(B)


# speedup vs baseline: 30.0400x; 2.8644x over previous
import functools

import jax
import jax.numpy as jnp
from jax import lax
from jax.experimental import pallas as pl
from jax.experimental.pallas import tpu as pltpu
from jax.experimental.pallas import tpu_sc as plsc

N = 16777216
B = 1048576

NC = 2
NS = 16
NW = NC * NS
BPW = B // NW
HALF = BPW // 2
NROW = NW * 2
CHUNK = 128
L = 16


def _dsu_body(father_hbm, mask_hbm, iota_hbm, out_hbm,
              idx_ref, pos_ref, g_ref, res_ref, sem):
    wid = lax.axis_index("s") * NC + lax.axis_index("c")

    for h in range(2):
        row = wid * 2 + h
        pltpu.sync_copy(mask_hbm.at[row], idx_ref)
        pltpu.sync_copy(iota_hbm, pos_ref)

        def step(n):
            nch = (n + CHUNK - 1) // CHUNK

            def fire(j, c):
                pltpu.make_async_copy(
                    father_hbm.at[idx_ref.at[pl.ds(j * CHUNK, CHUNK)]],
                    g_ref.at[pl.ds(j * CHUNK, CHUNK)],
                    sem,
                ).start()
                return c

            lax.fori_loop(0, nch, fire, 0)

            def drain(j, c):
                pltpu.make_async_copy(
                    father_hbm.at[idx_ref.at[pl.ds(j * CHUNK, CHUNK)]],
                    g_ref.at[pl.ds(j * CHUNK, CHUNK)],
                    sem,
                ).wait()
                return c

            lax.fori_loop(0, nch, drain, 0)

            ng = (n + L - 1) // L

            def proc(i, w):
                off = i * L
                g16 = g_ref[pl.ds(off, L)]
                i16 = idx_ref[pl.ds(off, L)]
                p16 = pos_ref[pl.ds(off, L)]
                valid = (lax.iota(jnp.int32, L) + off) < n
                eq = g16 == i16
                done = eq & valid
                act = (~eq) & valid
                plsc.store_scatter(res_ref, [p16], g16, mask=done)
                cnt = jnp.max(plsc.all_reduce_population_count(act))
                plsc.store_compressed(idx_ref.at[pl.ds(w, L)], g16, mask=act)
                plsc.store_compressed(pos_ref.at[pl.ds(w, L)], p16, mask=act)
                return w + cnt

            return lax.fori_loop(0, ng, proc, jnp.int32(0))

        lax.while_loop(lambda n: n > 0, step, jnp.int32(HALF))
        pltpu.sync_copy(res_ref, out_hbm.at[row])


_call = functools.partial(
    pl.kernel,
    out_type=jax.ShapeDtypeStruct((NROW, HALF), jnp.int32),
    mesh=plsc.VectorSubcoreMesh(core_axis_name="c", subcore_axis_name="s"),
    scratch_types=[
        pltpu.VMEM((HALF,), jnp.int32),
        pltpu.VMEM((HALF,), jnp.int32),
        pltpu.VMEM((HALF,), jnp.int32),
        pltpu.VMEM((HALF,), jnp.int32),
        pltpu.SemaphoreType.DMA,
    ],
    compiler_params=pltpu.CompilerParams(needs_layout_passes=False),
)(_dsu_body)


def kernel(father, mask):
    iota = jnp.arange(HALF, dtype=jnp.int32)
    out = _call(father, mask.reshape(NROW, HALF), iota)
    return out.reshape(B)


# speedup vs baseline: 36.8514x; 1.2267x over previous
import functools

import jax
import jax.numpy as jnp
from jax import lax
from jax.experimental import pallas as pl
from jax.experimental.pallas import tpu as pltpu
from jax.experimental.pallas import tpu_sc as plsc

N = 16777216
B = 1048576

NC = 2
NS = 16
NW = NC * NS
BPW = B // NW
QTR = BPW // 4
NROW = NW * 4
CHUNK = 128
L = 16


def _dsu_body(father_hbm, mask_hbm, iota_hbm, out_hbm,
              idx_a, pos_a, g_a, res_a, idx_b, pos_b, g_b, res_b,
              sem_a, sem_b):
    wid = lax.axis_index("s") * NC + lax.axis_index("c")

    def fire(idx_r, g_ref, sem, n):
        nch = (n + CHUNK - 1) // CHUNK

        def go(j, c):
            pltpu.make_async_copy(
                father_hbm.at[idx_r.at[pl.ds(j * CHUNK, CHUNK)]],
                g_ref.at[pl.ds(j * CHUNK, CHUNK)],
                sem,
            ).start()
            return c

        lax.fori_loop(0, nch, go, 0)

    def drain(idx_r, g_ref, sem, n):
        nch = (n + CHUNK - 1) // CHUNK

        def go(j, c):
            pltpu.make_async_copy(
                father_hbm.at[idx_r.at[pl.ds(j * CHUNK, CHUNK)]],
                g_ref.at[pl.ds(j * CHUNK, CHUNK)],
                sem,
            ).wait()
            return c

        lax.fori_loop(0, nch, go, 0)

    def proc(idx_r, pos_r, g_ref, res_ref, n):
        ng = (n + L - 1) // L

        def go(i, w):
            off = i * L
            g16 = g_ref[pl.ds(off, L)]
            i16 = idx_r[pl.ds(off, L)]
            p16 = pos_r[pl.ds(off, L)]
            valid = (lax.iota(jnp.int32, L) + off) < n
            eq = g16 == i16
            done = eq & valid
            act = (~eq) & valid
            plsc.store_scatter(res_ref, [p16], g16, mask=done)
            cnt = jnp.max(plsc.all_reduce_population_count(act))
            plsc.store_compressed(idx_r.at[pl.ds(w, L)], g16, mask=act)
            plsc.store_compressed(pos_r.at[pl.ds(w, L)], p16, mask=act)
            return w + cnt

        return lax.fori_loop(0, ng, go, jnp.int32(0))

    for pair in range(2):
        row_a = wid * 4 + 2 * pair
        row_b = row_a + 1
        pltpu.sync_copy(mask_hbm.at[row_a], idx_a)
        pltpu.sync_copy(iota_hbm, pos_a)
        pltpu.sync_copy(mask_hbm.at[row_b], idx_b)
        pltpu.sync_copy(iota_hbm, pos_b)
        fire(idx_a, g_a, sem_a, jnp.int32(QTR))
        fire(idx_b, g_b, sem_b, jnp.int32(QTR))

        def both(carry):
            na, nb = carry
            drain(idx_a, g_a, sem_a, na)
            na2 = proc(idx_a, pos_a, g_a, res_a, na)
            fire(idx_a, g_a, sem_a, na2)
            drain(idx_b, g_b, sem_b, nb)
            nb2 = proc(idx_b, pos_b, g_b, res_b, nb)
            fire(idx_b, g_b, sem_b, nb2)
            return na2, nb2

        lax.while_loop(
            lambda c: (c[0] > 0) | (c[1] > 0),
            both,
            (jnp.int32(QTR), jnp.int32(QTR)),
        )
        pltpu.sync_copy(res_a, out_hbm.at[row_a])
        pltpu.sync_copy(res_b, out_hbm.at[row_b])


_call = functools.partial(
    pl.kernel,
    out_type=jax.ShapeDtypeStruct((NROW, QTR), jnp.int32),
    mesh=plsc.VectorSubcoreMesh(core_axis_name="c", subcore_axis_name="s"),
    scratch_types=[
        pltpu.VMEM((QTR,), jnp.int32),
        pltpu.VMEM((QTR,), jnp.int32),
        pltpu.VMEM((QTR,), jnp.int32),
        pltpu.VMEM((QTR,), jnp.int32),
        pltpu.VMEM((QTR,), jnp.int32),
        pltpu.VMEM((QTR,), jnp.int32),
        pltpu.VMEM((QTR,), jnp.int32),
        pltpu.VMEM((QTR,), jnp.int32),
        pltpu.SemaphoreType.DMA,
        pltpu.SemaphoreType.DMA,
    ],
    compiler_params=pltpu.CompilerParams(needs_layout_passes=False),
)(_dsu_body)


def kernel(father, mask):
    iota = jnp.arange(QTR, dtype=jnp.int32)
    out = _call(father, mask.reshape(NROW, QTR), iota)
    return out.reshape(B)


# speedup vs baseline: 37.4594x; 1.0165x over previous
import functools

import jax
import jax.numpy as jnp
from jax import lax
from jax.experimental import pallas as pl
from jax.experimental.pallas import tpu as pltpu
from jax.experimental.pallas import tpu_sc as plsc

N = 16777216
B = 1048576

NC = 2
NS = 16
NW = NC * NS
BPW = B // NW
QTR = BPW // 4
NROW = NW * 4
CHUNK = 128
L = 16


def _dsu_body(father_hbm, mask_hbm, iota_hbm, out_hbm,
              idx_a, pos_a, g_a, res_a, idx_b, pos_b, g_b, res_b,
              sem_a, sem_b):
    wid = lax.axis_index("s") * NC + lax.axis_index("c")

    def fire(idx_r, g_ref, sem, n):
        nch = (n + CHUNK - 1) // CHUNK

        def go(j, c):
            pltpu.make_async_copy(
                father_hbm.at[idx_r.at[pl.ds(j * CHUNK, CHUNK)]],
                g_ref.at[pl.ds(j * CHUNK, CHUNK)],
                sem,
            ).start()
            return c

        lax.fori_loop(0, nch, go, 0)

    def drain(idx_r, g_ref, sem, n):
        nch = (n + CHUNK - 1) // CHUNK

        def go(j, c):
            pltpu.make_async_copy(
                father_hbm.at[idx_r.at[pl.ds(j * CHUNK, CHUNK)]],
                g_ref.at[pl.ds(j * CHUNK, CHUNK)],
                sem,
            ).wait()
            return c

        lax.fori_loop(0, nch, go, 0)

    def proc(idx_r, pos_r, g_ref, res_ref, n):
        nfull = n // L

        def step16(i, w, valid):
            off = i * L
            g16 = g_ref[pl.ds(off, L)]
            i16 = idx_r[pl.ds(off, L)]
            p16 = pos_r[pl.ds(off, L)]
            eq = g16 == i16
            if valid is None:
                done = eq
                act = ~eq
            else:
                done = eq & valid
                act = (~eq) & valid
            plsc.store_scatter(res_ref, [p16], g16, mask=done)
            cnt = plsc.all_reduce_population_count(act)[0]
            plsc.store_compressed(idx_r.at[pl.ds(w, L)], g16, mask=act)
            plsc.store_compressed(pos_r.at[pl.ds(w, L)], p16, mask=act)
            return w + cnt

        w = lax.fori_loop(0, nfull, lambda i, w: step16(i, w, None),
                          jnp.int32(0))

        def tail(_, w):
            valid = (lax.iota(jnp.int32, L) + nfull * L) < n
            return step16(nfull, w, valid)

        has_tail = (n % L != 0).astype(jnp.int32)
        return lax.fori_loop(0, has_tail, tail, w)

    for pair in range(2):
        row_a = wid * 4 + 2 * pair
        row_b = row_a + 1
        pltpu.sync_copy(mask_hbm.at[row_a], idx_a)
        pltpu.sync_copy(iota_hbm, pos_a)
        pltpu.sync_copy(mask_hbm.at[row_b], idx_b)
        pltpu.sync_copy(iota_hbm, pos_b)
        fire(idx_a, g_a, sem_a, jnp.int32(QTR))
        fire(idx_b, g_b, sem_b, jnp.int32(QTR))

        def both(carry):
            na, nb = carry
            drain(idx_a, g_a, sem_a, na)
            na2 = proc(idx_a, pos_a, g_a, res_a, na)
            fire(idx_a, g_a, sem_a, na2)
            drain(idx_b, g_b, sem_b, nb)
            nb2 = proc(idx_b, pos_b, g_b, res_b, nb)
            fire(idx_b, g_b, sem_b, nb2)
            return na2, nb2

        lax.while_loop(
            lambda c: (c[0] > 0) | (c[1] > 0),
            both,
            (jnp.int32(QTR), jnp.int32(QTR)),
        )
        pltpu.sync_copy(res_a, out_hbm.at[row_a])
        pltpu.sync_copy(res_b, out_hbm.at[row_b])


_call = functools.partial(
    pl.kernel,
    out_type=jax.ShapeDtypeStruct((NROW, QTR), jnp.int32),
    mesh=plsc.VectorSubcoreMesh(core_axis_name="c", subcore_axis_name="s"),
    scratch_types=[
        pltpu.VMEM((QTR,), jnp.int32),
        pltpu.VMEM((QTR,), jnp.int32),
        pltpu.VMEM((QTR,), jnp.int32),
        pltpu.VMEM((QTR,), jnp.int32),
        pltpu.VMEM((QTR,), jnp.int32),
        pltpu.VMEM((QTR,), jnp.int32),
        pltpu.VMEM((QTR,), jnp.int32),
        pltpu.VMEM((QTR,), jnp.int32),
        pltpu.SemaphoreType.DMA,
        pltpu.SemaphoreType.DMA,
    ],
    compiler_params=pltpu.CompilerParams(needs_layout_passes=False),
)(_dsu_body)


def kernel(father, mask):
    iota = jnp.arange(QTR, dtype=jnp.int32)
    out = _call(father, mask.reshape(NROW, QTR), iota)
    return out.reshape(B)


# speedup vs baseline: 37.4776x; 1.0005x over previous
import functools

import jax
import jax.numpy as jnp
from jax import lax
from jax.experimental import pallas as pl
from jax.experimental.pallas import tpu as pltpu
from jax.experimental.pallas import tpu_sc as plsc

N = 16777216
B = 1048576

NC = 2
NS = 16
NW = NC * NS
BPW = B // NW
QTR = BPW // 4
NROW = NW * 4
CHUNK = 128
L = 16
GPC = CHUNK // L
S = 4


def _dsu_body(father_hbm, mask_hbm, iota_hbm, out_hbm,
              idx_a, pos_a, g_a, res_a, idx_b, pos_b, g_b, res_b,
              idx_c, g_c, idx_d, g_d, *sems):
    wid = lax.axis_index("s") * NC + lax.axis_index("c")
    sems_of = {
        "a": sems[0:S], "b": sems[S:2 * S],
        "c": sems[2 * S:3 * S], "d": sems[3 * S:4 * S],
    }

    def subrange(n, s):
        nch = (n + CHUNK - 1) // CHUNK
        q = (nch + S - 1) // S
        cs = jnp.minimum(s * q, nch)
        ce = jnp.minimum((s + 1) * q, nch)
        return cs, ce

    def fire(idx_r, g_ref, sem4, n):
        for s in range(S):
            cs, ce = subrange(n, s)

            def go(j, c):
                pltpu.make_async_copy(
                    father_hbm.at[idx_r.at[pl.ds(j * CHUNK, CHUNK)]],
                    g_ref.at[pl.ds(j * CHUNK, CHUNK)],
                    sem4[s],
                ).start()
                return c

            lax.fori_loop(cs, ce, go, 0)

    def drain_proc(idx_r, pos_r, g_ref, res_ref, sem4, n):
        ng = (n + L - 1) // L
        w = jnp.int32(0)
        for s in range(S):
            cs, ce = subrange(n, s)

            def dgo(j, c):
                pltpu.make_async_copy(
                    father_hbm.at[idx_r.at[pl.ds(j * CHUNK, CHUNK)]],
                    g_ref.at[pl.ds(j * CHUNK, CHUNK)],
                    sem4[s],
                ).wait()
                return c

            lax.fori_loop(cs, ce, dgo, 0)

            def proc(i, w):
                off = i * L
                g16 = g_ref[pl.ds(off, L)]
                i16 = idx_r[pl.ds(off, L)]
                p16 = pos_r[pl.ds(off, L)]
                valid = (lax.iota(jnp.int32, L) + off) < n
                eq = g16 == i16
                done = eq & valid
                act = (~eq) & valid
                plsc.store_scatter(res_ref, [p16], g16, mask=done)
                cnt = plsc.all_reduce_population_count(act)[0]
                plsc.store_compressed(idx_r.at[pl.ds(w, L)], g16, mask=act)
                plsc.store_compressed(pos_r.at[pl.ds(w, L)], p16, mask=act)
                return w + cnt

            w = lax.fori_loop(cs * GPC, jnp.minimum(ce * GPC, ng), proc, w)
        return w

    def run_pair(idx_1, pos_1, g_1, res_1, k1,
                 idx_2, pos_2, g_2, res_2, k2, row_1, row_2,
                 prefired):
        pltpu.sync_copy(iota_hbm, pos_1)
        pltpu.sync_copy(iota_hbm, pos_2)
        if not prefired:
            pltpu.sync_copy(mask_hbm.at[row_1], idx_1)
            pltpu.sync_copy(mask_hbm.at[row_2], idx_2)
            fire(idx_1, g_1, sems_of[k1], jnp.int32(QTR))
            fire(idx_2, g_2, sems_of[k2], jnp.int32(QTR))

        def both(carry):
            n1, n2 = carry
            n1n = drain_proc(idx_1, pos_1, g_1, res_1, sems_of[k1], n1)
            fire(idx_1, g_1, sems_of[k1], n1n)
            n2n = drain_proc(idx_2, pos_2, g_2, res_2, sems_of[k2], n2)
            fire(idx_2, g_2, sems_of[k2], n2n)
            return n1n, n2n

        lax.while_loop(
            lambda c: (c[0] > 0) | (c[1] > 0),
            both,
            (jnp.int32(QTR), jnp.int32(QTR)),
        )
        pltpu.sync_copy(res_1, out_hbm.at[row_1])
        pltpu.sync_copy(res_2, out_hbm.at[row_2])

    base = wid * 4
    pltpu.sync_copy(mask_hbm.at[base + 2], idx_c)
    pltpu.sync_copy(mask_hbm.at[base + 3], idx_d)
    fire(idx_c, g_c, sems_of["c"], jnp.int32(QTR))
    fire(idx_d, g_d, sems_of["d"], jnp.int32(QTR))

    run_pair(idx_a, pos_a, g_a, res_a, "a",
             idx_b, pos_b, g_b, res_b, "b", base, base + 1, False)
    run_pair(idx_c, pos_a, g_c, res_a, "c",
             idx_d, pos_b, g_d, res_b, "d", base + 2, base + 3, True)


_call = functools.partial(
    pl.kernel,
    out_type=jax.ShapeDtypeStruct((NROW, QTR), jnp.int32),
    mesh=plsc.VectorSubcoreMesh(core_axis_name="c", subcore_axis_name="s"),
    scratch_types=(
        [pltpu.VMEM((QTR,), jnp.int32)] * 12
        + [pltpu.SemaphoreType.DMA] * (4 * S)
    ),
    compiler_params=pltpu.CompilerParams(needs_layout_passes=False),
)(_dsu_body)


def kernel(father, mask):
    iota = jnp.arange(QTR, dtype=jnp.int32)
    out = _call(father, mask.reshape(NROW, QTR), iota)
    return out.reshape(B)
